# baseline (speedup 1.0000x reference)
import functools

import jax
import jax.numpy as jnp
from jax import lax
from jax.experimental import pallas as pl
from jax.experimental.pallas import tpu as pltpu
from jax.experimental.pallas import tpu_sc as plsc

NC = 2
NS = 16
EB = 64
D = 128
RELS = 3


def _sc_mesh():
    return plsc.VectorSubcoreMesh(core_axis_name="c", subcore_axis_name="s")


def _scatter_kernel(t_pad, blkmax):
    stripe = t_pad // NS

    def body(h_hbm, sf_hbm, nblk_hbm, zeros_hbm, slo_hbm, shi_hbm,
             acc_sh, nv):
        cid = lax.axis_index("c")
        sid = lax.axis_index("s")

        def run(out_hbm):
            pltpu.sync_copy(
                zeros_hbm, acc_sh.at[pl.ds(sid * stripe, stripe)])
            pltpu.sync_copy(nblk_hbm.at[cid], nv)
            nb = nv[...][0]
            plsc.subcore_barrier()

            @plsc.parallel_loop(0, nb, unroll=2)
            def _(j):
                def scoped(idx_v, rows_v, sem):
                    pltpu.sync_copy(sf_hbm.at[cid, sid, j], idx_v)
                    pltpu.async_copy(
                        h_hbm.at[idx_v.at[0]], rows_v, sem).wait()
                    pltpu.sync_copy(
                        rows_v, acc_sh.at[idx_v.at[1]], add=True)
                pl.run_scoped(
                    scoped,
                    pltpu.VMEM((2, EB), jnp.int32),
                    pltpu.VMEM((EB, D), jnp.float32),
                    pltpu.SemaphoreType.DMA)

            plsc.subcore_barrier()
            pltpu.sync_copy(acc_sh.at[pl.ds(sid * stripe, stripe)],
                            out_hbm.at[pl.ds(sid * stripe, stripe)])

        @pl.when(cid == 0)
        def _():
            run(slo_hbm)

        @pl.when(cid == 1)
        def _():
            run(shi_hbm)

    return pl.kernel(
        body,
        out_type=(jax.ShapeDtypeStruct((t_pad, D), jnp.float32),
                  jax.ShapeDtypeStruct((t_pad, D), jnp.float32)),
        mesh=_sc_mesh(),
        scratch_types=[
            pltpu.VMEM_SHARED((t_pad, D), jnp.float32),
            pltpu.VMEM((16,), jnp.int32),
        ],
        compiler_params=pltpu.CompilerParams(use_tc_tiling_on_sc=False),
    )


def _counts_kernel(n_fused, blk):
    stripe = n_fused // NS

    def body(sf_hbm, ones_hbm, zeros_hbm, cnt_hbm, idx_v, ones_v, acc_sh, sem):
        cid = lax.axis_index("c")
        sid = lax.axis_index("s")

        @pl.when(cid == 0)
        def _():
            pltpu.sync_copy(
                zeros_hbm, acc_sh.at[pl.ds(sid * stripe, stripe)])
            pltpu.sync_copy(ones_hbm, ones_v)
            plsc.subcore_barrier()

            @pl.loop(0, blk)
            def _(j):
                pltpu.sync_copy(sf_hbm.at[sid, j], idx_v)
                pltpu.sync_copy(ones_v, acc_sh.at[idx_v.at[1]], add=True)

            plsc.subcore_barrier()
            pltpu.sync_copy(acc_sh.at[pl.ds(sid * stripe, stripe)],
                            cnt_hbm.at[pl.ds(sid * stripe, stripe)])

    return pl.kernel(
        body,
        out_type=jax.ShapeDtypeStruct((n_fused, 16), jnp.float32),
        mesh=_sc_mesh(),
        scratch_types=[
            pltpu.VMEM((2, EB), jnp.int32),
            pltpu.VMEM((EB, 16), jnp.float32),
            pltpu.VMEM_SHARED((n_fused, 16), jnp.float32),
            pltpu.SemaphoreType.DMA,
        ],
        compiler_params=pltpu.CompilerParams(use_tc_tiling_on_sc=False),
    )


def _tc_layer_body(gated, s_ref, inv_ref, h_ref, w_ref, wg_ref, bg_ref,
                   o_ref):
    f32 = jnp.float32
    a = jnp.dot(s_ref[...] * inv_ref[...], w_ref[...],
                preferred_element_type=f32)
    if gated:
        g = jnp.dot(h_ref[...], wg_ref[...], preferred_element_type=f32)
        g = jax.nn.sigmoid(g + bg_ref[...])
        out = jnp.maximum(g * a, 0.0)
    else:
        out = a - jnp.tanh(a)
    o_ref[...] = out


def _tc_layer(n_pad, gated, bn):
    kdim = RELS * D
    grid = (n_pad // bn,)
    row_blk = lambda w: pl.BlockSpec((bn, w), lambda i: (i, 0))
    full = lambda a, b: pl.BlockSpec((a, b), lambda i: (0, 0))
    return pl.pallas_call(
        functools.partial(_tc_layer_body, gated),
        grid=grid,
        in_specs=[
            row_blk(kdim), row_blk(kdim), row_blk(D),
            full(kdim, D), full(D, D), full(1, D),
        ],
        out_specs=row_blk(D),
        out_shape=jax.ShapeDtypeStruct((n_pad, D), jnp.float32),
    )


def kernel(x, edge_index, edge_type, enc_W, enc_Wg, enc_bg, dec_W):
    n, d = x.shape
    e = edge_index.shape[1]
    assert d == D

    n_pad = ((n + 127) // 128) * 128
    bn = n_pad // 8
    n_fused = n_pad * RELS
    h_split = (n + 1) // 2
    t_real = h_split * RELS
    t_pad = NS * (-(-(t_real + 1) // NS))
    stripe = t_pad // NS
    blkmax = 2 * (-(-e // (NS * EB * 2)))

    src = edge_index[0]
    dst = edge_index[1]
    et = edge_type

    half = (dst >= h_split).astype(jnp.int32)
    order = jnp.argsort(half * (1 << 24) + src)
    s_src = src[order]
    s_half = half[order]
    s_local = ((dst[order] - s_half * h_split) * RELS + edge_type[order])
    e1 = jnp.sum(s_half)
    e0 = e - e1
    tile_cap = NS * EB

    def blocks_even(cnt):
        b = (cnt + tile_cap - 1) // tile_cap
        return ((b + 1) // 2) * 2

    nb0, nb1 = blocks_even(e0), blocks_even(e1)

    p = jnp.arange(e, dtype=jnp.int32) - s_half * e0
    nbc = jnp.maximum(jnp.where(s_half == 1, nb1, nb0), 1) * EB
    slot = (s_half * NS + p // nbc) * (blkmax * EB) + p % nbc
    src_buf = jnp.zeros((NC * NS * blkmax * EB,), jnp.int32).at[slot].set(s_src)
    loc_buf = jnp.full((NC * NS * blkmax * EB,), t_real,
                       jnp.int32).at[slot].set(s_local)
    sf_p = jnp.concatenate(
        [src_buf.reshape(NC, NS, blkmax, 1, EB),
         loc_buf.reshape(NC, NS, blkmax, 1, EB)], axis=3)
    nblk = jnp.tile(jnp.stack([nb0, nb1]).astype(jnp.int32)[:, None], (1, 16))

    fused_g = jnp.concatenate(
        [dst * RELS + et,
         jnp.full((NS * EB * blkmax - e,), n * RELS, jnp.int32)])
    cnt_sf = jnp.concatenate(
        [jnp.zeros((NS, blkmax, 1, EB), jnp.int32),
         fused_g.reshape(NS, blkmax, 1, EB)], axis=2)
    zeros16 = jnp.zeros((n_fused // NS, 16), jnp.float32)
    ones16 = jnp.ones((EB, 16), jnp.float32)
    cnt = _counts_kernel(n_fused, blkmax)(cnt_sf, ones16, zeros16)
    inv = 1.0 / jnp.maximum(cnt[:, 0], 1.0)
    inv_e = jnp.repeat(inv.reshape(n_pad, RELS), D, axis=1)

    scatter = _scatter_kernel(t_pad, blkmax)
    zeros_st = jnp.zeros((stripe, D), jnp.float32)
    zpad = jnp.zeros((n_fused - 2 * t_real, D), jnp.float32)

    h = jnp.zeros((n_pad, d), x.dtype).at[:n].set(x)

    def agg(h):
        s_lo, s_hi = scatter(h, sf_p, nblk, zeros_st)
        s = jnp.concatenate([s_lo[:t_real], s_hi[:t_real], zpad], axis=0)
        return s.reshape(n_pad, RELS * D)

    enc = _tc_layer(n_pad, gated=True, bn=bn)
    dec = _tc_layer(n_pad, gated=False, bn=bn)
    zg = jnp.zeros((D, D), jnp.float32)
    zb = jnp.zeros((1, D), jnp.float32)

    for l in range(enc_W.shape[0]):
        h = enc(agg(h), inv_e, h, enc_W[l].reshape(RELS * D, D),
                enc_Wg[l], enc_bg[l].reshape(1, D))
    for l in range(dec_W.shape[0]):
        h = dec(agg(h), inv_e, h, dec_W[l].reshape(RELS * D, D), zg, zb)

    return h[:n]


# speedup vs baseline: 1.6244x; 1.6244x over previous
import functools

import jax
import jax.numpy as jnp
from jax import lax
from jax.experimental import pallas as pl
from jax.experimental.pallas import tpu as pltpu
from jax.experimental.pallas import tpu_sc as plsc

NC = 2
NS = 16
EB = 128
DH = 64
RELS = 3


def _sc_mesh():
    return plsc.VectorSubcoreMesh(core_axis_name="c", subcore_axis_name="s")


def _scatter_kernel(n_fused, blk):
    stripe = n_fused // NS
    assert blk % 2 == 0

    def body(h0_hbm, h1_hbm, sf_hbm, zeros_hbm, s0_hbm, s1_hbm, acc_sh):
        cid = lax.axis_index("c")
        sid = lax.axis_index("s")

        def run(h_hbm, out_hbm):
            pltpu.sync_copy(
                zeros_hbm, acc_sh.at[pl.ds(sid * stripe, stripe)])
            plsc.subcore_barrier()

            @plsc.parallel_loop(0, blk, unroll=2)
            def _(j):
                def scoped(idx_v, rows_v, sem):
                    pltpu.sync_copy(sf_hbm.at[sid, j], idx_v)
                    pltpu.async_copy(
                        h_hbm.at[idx_v.at[0]], rows_v, sem).wait()
                    pltpu.sync_copy(
                        rows_v, acc_sh.at[idx_v.at[1]], add=True)
                pl.run_scoped(
                    scoped,
                    pltpu.VMEM((2, EB), jnp.int32),
                    pltpu.VMEM((EB, DH), jnp.float32),
                    pltpu.SemaphoreType.DMA)

            plsc.subcore_barrier()
            pltpu.sync_copy(acc_sh.at[pl.ds(sid * stripe, stripe)],
                            out_hbm.at[pl.ds(sid * stripe, stripe)])

        @pl.when(cid == 0)
        def _():
            run(h0_hbm, s0_hbm)

        @pl.when(cid == 1)
        def _():
            run(h1_hbm, s1_hbm)

    return pl.kernel(
        body,
        out_type=(jax.ShapeDtypeStruct((n_fused, DH), jnp.float32),
                  jax.ShapeDtypeStruct((n_fused, DH), jnp.float32)),
        mesh=_sc_mesh(),
        scratch_types=[
            pltpu.VMEM_SHARED((n_fused, DH), jnp.float32),
        ],
        compiler_params=pltpu.CompilerParams(use_tc_tiling_on_sc=False),
    )


def _counts_kernel(n_fused, blk):
    stripe = n_fused // NS

    def body(sf_hbm, ones_hbm, zeros_hbm, cnt_hbm, idx_v, ones_v, acc_sh, sem):
        cid = lax.axis_index("c")
        sid = lax.axis_index("s")

        @pl.when(cid == 0)
        def _():
            pltpu.sync_copy(
                zeros_hbm, acc_sh.at[pl.ds(sid * stripe, stripe)])
            pltpu.sync_copy(ones_hbm, ones_v)
            plsc.subcore_barrier()

            @pl.loop(0, blk)
            def _(j):
                pltpu.sync_copy(sf_hbm.at[sid, j], idx_v)
                pltpu.sync_copy(ones_v, acc_sh.at[idx_v.at[1]], add=True)

            plsc.subcore_barrier()
            pltpu.sync_copy(acc_sh.at[pl.ds(sid * stripe, stripe)],
                            cnt_hbm.at[pl.ds(sid * stripe, stripe)])

    return pl.kernel(
        body,
        out_type=jax.ShapeDtypeStruct((n_fused, 16), jnp.float32),
        mesh=_sc_mesh(),
        scratch_types=[
            pltpu.VMEM((2, EB), jnp.int32),
            pltpu.VMEM((EB, 16), jnp.float32),
            pltpu.VMEM_SHARED((n_fused, 16), jnp.float32),
            pltpu.SemaphoreType.DMA,
        ],
        compiler_params=pltpu.CompilerParams(use_tc_tiling_on_sc=False),
    )


def _tc_layer_body(gated, s0_ref, s1_ref, inv_ref, h0_ref, h1_ref,
                   wa_ref, wb_ref, wg0_ref, wg1_ref, bg_ref,
                   o0_ref, o1_ref):
    f32 = jnp.float32
    a = jnp.dot(s0_ref[...] * inv_ref[...], wa_ref[...],
                preferred_element_type=f32)
    a = a + jnp.dot(s1_ref[...] * inv_ref[...], wb_ref[...],
                    preferred_element_type=f32)
    if gated:
        g = jnp.dot(h0_ref[...], wg0_ref[...], preferred_element_type=f32)
        g = g + jnp.dot(h1_ref[...], wg1_ref[...], preferred_element_type=f32)
        g = jax.nn.sigmoid(g + bg_ref[...])
        out = jnp.maximum(g * a, 0.0)
    else:
        out = a - jnp.tanh(a)
    o0_ref[...] = out[:, :DH]
    o1_ref[...] = out[:, DH:]


def _tc_layer(n_pad, gated, bn):
    kdim = RELS * DH
    grid = (n_pad // bn,)
    row_blk = lambda w: pl.BlockSpec((bn, w), lambda i: (i, 0))
    full = lambda a, b: pl.BlockSpec((a, b), lambda i: (0, 0))
    return pl.pallas_call(
        functools.partial(_tc_layer_body, gated),
        grid=grid,
        in_specs=[
            row_blk(kdim), row_blk(kdim), row_blk(kdim),
            row_blk(DH), row_blk(DH),
            full(kdim, 2 * DH), full(kdim, 2 * DH),
            full(DH, 2 * DH), full(DH, 2 * DH), full(1, 2 * DH),
        ],
        out_specs=[row_blk(DH), row_blk(DH)],
        out_shape=(jax.ShapeDtypeStruct((n_pad, DH), jnp.float32),
                   jax.ShapeDtypeStruct((n_pad, DH), jnp.float32)),
    )


def kernel(x, edge_index, edge_type, enc_W, enc_Wg, enc_bg, dec_W):
    n, d = x.shape
    e = edge_index.shape[1]
    assert d == 2 * DH

    n_pad = 10112 if n <= 10112 else ((n + 15) // 16) * 16
    bn = n_pad // 8
    n_fused = n_pad * RELS
    stripe = n_fused // NS
    blk = 2 * (-(-e // (NS * EB * 2)))
    e_pad = NS * EB * blk

    order = jnp.argsort(edge_index[0])
    src = edge_index[0][order]
    dst = edge_index[1][order]
    fused = dst * RELS + edge_type[order]
    pad = e_pad - e
    src_p = jnp.concatenate(
        [src, jnp.zeros((pad,), jnp.int32)]).reshape(NS, blk, 1, EB)
    fused_p = jnp.concatenate(
        [fused, jnp.full((pad,), n * RELS, jnp.int32)]).reshape(NS, blk, 1, EB)
    sf_p = jnp.concatenate([src_p, fused_p], axis=2)

    zeros_st = jnp.zeros((stripe, DH), jnp.float32)
    zeros16 = jnp.zeros((stripe, 16), jnp.float32)
    ones16 = jnp.ones((EB, 16), jnp.float32)
    scatter = _scatter_kernel(n_fused, blk)

    cnt = _counts_kernel(n_fused, blk)(sf_p, ones16, zeros16)
    inv = 1.0 / jnp.maximum(cnt[:, 0], 1.0)
    inv_e = jnp.repeat(inv.reshape(n_pad, RELS), DH, axis=1)

    x_pad = jnp.zeros((n_pad, d), x.dtype).at[:n].set(x)
    h0, h1 = x_pad[:, :DH], x_pad[:, DH:]

    def agg_inputs(h0, h1, W):
        s0, s1 = scatter(h0, h1, sf_p, zeros_st)
        wa = W[:, :DH, :].reshape(RELS * DH, d)
        wb = W[:, DH:, :].reshape(RELS * DH, d)
        return (s0.reshape(n_pad, RELS * DH), s1.reshape(n_pad, RELS * DH),
                inv_e, h0, h1, wa, wb)

    enc = _tc_layer(n_pad, gated=True, bn=bn)
    dec = _tc_layer(n_pad, gated=False, bn=bn)
    zg = jnp.zeros((DH, d), jnp.float32)
    zb = jnp.zeros((1, d), jnp.float32)

    for l in range(enc_W.shape[0]):
        wg = enc_Wg[l]
        h0, h1 = enc(*agg_inputs(h0, h1, enc_W[l]),
                     wg[:DH], wg[DH:], enc_bg[l].reshape(1, d))
    for l in range(dec_W.shape[0]):
        h0, h1 = dec(*agg_inputs(h0, h1, dec_W[l]), zg, zg, zb)

    return jnp.concatenate([h0, h1], axis=1)[:n]


# speedup vs baseline: 1.6253x; 1.0005x over previous
import functools

import jax
import jax.numpy as jnp
from jax import lax
from jax.experimental import pallas as pl
from jax.experimental.pallas import tpu as pltpu
from jax.experimental.pallas import tpu_sc as plsc

NC = 2
NS = 16
EB = 128
DH = 64
RELS = 3


def _sc_mesh():
    return plsc.VectorSubcoreMesh(core_axis_name="c", subcore_axis_name="s")


def _scatter_kernel(n_fused, blk):
    stripe = n_fused // NS

    def body(h0_hbm, h1_hbm, sf_hbm, zeros_hbm, s0_hbm, s1_hbm, acc_sh):
        cid = lax.axis_index("c")
        sid = lax.axis_index("s")

        def run(h_hbm, out_hbm):
            pltpu.sync_copy(
                zeros_hbm, acc_sh.at[pl.ds(sid * stripe, stripe)])
            plsc.subcore_barrier()

            @pl.loop(0, blk)
            def _(j):
                def scoped(idx_v, rows_v, sem):
                    pltpu.sync_copy(sf_hbm.at[sid, j], idx_v)
                    pltpu.async_copy(
                        h_hbm.at[idx_v.at[0]], rows_v, sem).wait()
                    pltpu.sync_copy(
                        rows_v, acc_sh.at[idx_v.at[1]], add=True)
                pl.run_scoped(
                    scoped,
                    pltpu.VMEM((2, EB), jnp.int32),
                    pltpu.VMEM((EB, DH), jnp.float32),
                    pltpu.SemaphoreType.DMA)

            plsc.subcore_barrier()
            pltpu.sync_copy(acc_sh.at[pl.ds(sid * stripe, stripe)],
                            out_hbm.at[pl.ds(sid * stripe, stripe)])

        @pl.when(cid == 0)
        def _():
            run(h0_hbm, s0_hbm)

        @pl.when(cid == 1)
        def _():
            run(h1_hbm, s1_hbm)

    return pl.kernel(
        body,
        out_type=(jax.ShapeDtypeStruct((n_fused, DH), jnp.float32),
                  jax.ShapeDtypeStruct((n_fused, DH), jnp.float32)),
        mesh=_sc_mesh(),
        scratch_types=[
            pltpu.VMEM_SHARED((n_fused, DH), jnp.float32),
        ],
        compiler_params=pltpu.CompilerParams(use_tc_tiling_on_sc=False),
    )


def _counts_kernel(n_fused, blk):
    stripe = n_fused // NS

    def body(sf_hbm, ones_hbm, zeros_hbm, cnt_hbm, idx_v, ones_v, acc_sh, sem):
        cid = lax.axis_index("c")
        sid = lax.axis_index("s")

        @pl.when(cid == 0)
        def _():
            pltpu.sync_copy(
                zeros_hbm, acc_sh.at[pl.ds(sid * stripe, stripe)])
            pltpu.sync_copy(ones_hbm, ones_v)
            plsc.subcore_barrier()

            @pl.loop(0, blk)
            def _(j):
                pltpu.sync_copy(sf_hbm.at[sid, j], idx_v)
                pltpu.sync_copy(ones_v, acc_sh.at[idx_v.at[1]], add=True)

            plsc.subcore_barrier()
            pltpu.sync_copy(acc_sh.at[pl.ds(sid * stripe, stripe)],
                            cnt_hbm.at[pl.ds(sid * stripe, stripe)])

    return pl.kernel(
        body,
        out_type=jax.ShapeDtypeStruct((n_fused, 16), jnp.float32),
        mesh=_sc_mesh(),
        scratch_types=[
            pltpu.VMEM((2, EB), jnp.int32),
            pltpu.VMEM((EB, 16), jnp.float32),
            pltpu.VMEM_SHARED((n_fused, 16), jnp.float32),
            pltpu.SemaphoreType.DMA,
        ],
        compiler_params=pltpu.CompilerParams(use_tc_tiling_on_sc=False),
    )


def _tc_layer_body(gated, s0_ref, s1_ref, inv_ref, h0_ref, h1_ref,
                   wa_ref, wb_ref, wg0_ref, wg1_ref, bg_ref,
                   o0_ref, o1_ref):
    f32 = jnp.float32
    a = jnp.dot(s0_ref[...] * inv_ref[...], wa_ref[...],
                preferred_element_type=f32)
    a = a + jnp.dot(s1_ref[...] * inv_ref[...], wb_ref[...],
                    preferred_element_type=f32)
    if gated:
        g = jnp.dot(h0_ref[...], wg0_ref[...], preferred_element_type=f32)
        g = g + jnp.dot(h1_ref[...], wg1_ref[...], preferred_element_type=f32)
        g = jax.nn.sigmoid(g + bg_ref[...])
        out = jnp.maximum(g * a, 0.0)
    else:
        out = a - jnp.tanh(a)
    o0_ref[...] = out[:, :DH]
    o1_ref[...] = out[:, DH:]


def _tc_layer(n_pad, gated, bn):
    kdim = RELS * DH
    grid = (n_pad // bn,)
    row_blk = lambda w: pl.BlockSpec((bn, w), lambda i: (i, 0))
    full = lambda a, b: pl.BlockSpec((a, b), lambda i: (0, 0))
    return pl.pallas_call(
        functools.partial(_tc_layer_body, gated),
        grid=grid,
        in_specs=[
            row_blk(kdim), row_blk(kdim), row_blk(kdim),
            row_blk(DH), row_blk(DH),
            full(kdim, 2 * DH), full(kdim, 2 * DH),
            full(DH, 2 * DH), full(DH, 2 * DH), full(1, 2 * DH),
        ],
        out_specs=[row_blk(DH), row_blk(DH)],
        out_shape=(jax.ShapeDtypeStruct((n_pad, DH), jnp.float32),
                   jax.ShapeDtypeStruct((n_pad, DH), jnp.float32)),
    )


def kernel(x, edge_index, edge_type, enc_W, enc_Wg, enc_bg, dec_W):
    n, d = x.shape
    e = edge_index.shape[1]
    assert d == 2 * DH

    n_pad = 10112 if n <= 10112 else ((n + 15) // 16) * 16
    bn = n_pad // 8
    n_fused = n_pad * RELS
    stripe = n_fused // NS
    blk = 2 * (-(-e // (NS * EB * 2)))
    e_pad = NS * EB * blk

    order = jnp.argsort(edge_index[0])
    src = edge_index[0][order]
    dst = edge_index[1][order]
    fused = dst * RELS + edge_type[order]
    pad = e_pad - e
    src_p = jnp.concatenate(
        [src, jnp.zeros((pad,), jnp.int32)]).reshape(NS, blk, 1, EB)
    fused_p = jnp.concatenate(
        [fused, jnp.full((pad,), n * RELS, jnp.int32)]).reshape(NS, blk, 1, EB)
    sf_p = jnp.concatenate([src_p, fused_p], axis=2)

    zeros_st = jnp.zeros((stripe, DH), jnp.float32)
    zeros16 = jnp.zeros((stripe, 16), jnp.float32)
    ones16 = jnp.ones((EB, 16), jnp.float32)
    scatter = _scatter_kernel(n_fused, blk)

    cnt = _counts_kernel(n_fused, blk)(sf_p, ones16, zeros16)
    inv = 1.0 / jnp.maximum(cnt[:, 0], 1.0)
    inv_e = jnp.repeat(inv.reshape(n_pad, RELS), DH, axis=1)

    x_pad = jnp.zeros((n_pad, d), x.dtype).at[:n].set(x)
    h0, h1 = x_pad[:, :DH], x_pad[:, DH:]

    def agg_inputs(h0, h1, W):
        s0, s1 = scatter(h0, h1, sf_p, zeros_st)
        wa = W[:, :DH, :].reshape(RELS * DH, d)
        wb = W[:, DH:, :].reshape(RELS * DH, d)
        return (s0.reshape(n_pad, RELS * DH), s1.reshape(n_pad, RELS * DH),
                inv_e, h0, h1, wa, wb)

    enc = _tc_layer(n_pad, gated=True, bn=bn)
    dec = _tc_layer(n_pad, gated=False, bn=bn)
    zg = jnp.zeros((DH, d), jnp.float32)
    zb = jnp.zeros((1, d), jnp.float32)

    for l in range(enc_W.shape[0]):
        wg = enc_Wg[l]
        h0, h1 = enc(*agg_inputs(h0, h1, enc_W[l]),
                     wg[:DH], wg[DH:], enc_bg[l].reshape(1, d))
    for l in range(dec_W.shape[0]):
        h0, h1 = dec(*agg_inputs(h0, h1, dec_W[l]), zg, zg, zb)

    return jnp.concatenate([h0, h1], axis=1)[:n]
